# Initial kernel scaffold; baseline (speedup 1.0000x reference)
import jax
import jax.numpy as jnp
from jax.experimental import pallas as pl


def kernel(x, batch, W_gnn, b_gnn, W, b):
    raise NotImplementedError("write your pallas kernel here")


# speedup vs baseline: 2.6636x; 2.6636x over previous
import functools

import jax
import jax.numpy as jnp
from jax import lax
from jax.experimental import pallas as pl
from jax.experimental.pallas import tpu as pltpu
from jax.experimental.pallas import tpu_sc as plsc

NUM_GRAPHS = 512
CHUNK = 128
GROUP = 2
BLK = 1024
NC, NS = 2, 16


def _gnn_matmul(x, w, bvec, n_pad):
    n, d = x.shape

    def body(x_ref, w_ref, b_ref, o_ref):
        acc = lax.dot_general(
            x_ref[...], w_ref[...], (((1,), (0,)), ((), ())),
            precision=lax.Precision.HIGHEST,
            preferred_element_type=jnp.float32)
        o_ref[...] = jnp.maximum(acc + b_ref[...], 0.0)

    return pl.pallas_call(
        body,
        grid=(n_pad // BLK,),
        in_specs=[
            pl.BlockSpec((BLK, d), lambda i: (i, 0)),
            pl.BlockSpec((d, d), lambda i: (0, 0)),
            pl.BlockSpec((1, d), lambda i: (0, 0)),
        ],
        out_specs=pl.BlockSpec((BLK, d), lambda i: (i, 0)),
        out_shape=jax.ShapeDtypeStruct((n_pad, d), jnp.float32),
    )(x, w, bvec.reshape(1, d))


def _sc_segment_sum(node_rep, ids2d, zeros_sum, ones):
    d = node_rep.shape[1]
    n_groups = ids2d.shape[0] // GROUP
    nw = NC * NS
    mesh = plsc.VectorSubcoreMesh(core_axis_name="c", subcore_axis_name="s")

    @functools.partial(
        pl.kernel,
        out_type=[
            jax.ShapeDtypeStruct((NC, NUM_GRAPHS, d), jnp.float32),
            jax.ShapeDtypeStruct((NC, NUM_GRAPHS, d), jnp.float32),
        ],
        mesh=mesh,
        scratch_types=[
            pltpu.VMEM((GROUP, CHUNK), jnp.int32),
            pltpu.VMEM((GROUP * CHUNK, d), jnp.float32),
            pltpu.VMEM((CHUNK, d), jnp.float32),
            pltpu.VMEM_SHARED((NUM_GRAPHS + 1, d), jnp.float32),
            pltpu.VMEM_SHARED((NUM_GRAPHS + 1, d), jnp.float32),
        ],
    )
    def k(rep_hbm, ids_hbm, z_sum_hbm, ones_hbm, out_sum, out_cnt,
          idx_v, rows_v, ones_v, acc_sh, cnt_sh):
        c = lax.axis_index("c")
        s = lax.axis_index("s")
        wid = s * NC + c

        @pl.when(s == 0)
        def _zero():
            pltpu.sync_copy(z_sum_hbm, acc_sh)
            pltpu.sync_copy(z_sum_hbm, cnt_sh)

        pltpu.sync_copy(ones_hbm, ones_v)
        plsc.subcore_barrier()

        n_mine = (n_groups - wid + nw - 1) // nw

        def body(g, carry):
            grp = wid + g * nw
            pltpu.sync_copy(ids_hbm.at[pl.ds(grp * GROUP, GROUP)], idx_v)
            pltpu.sync_copy(
                rep_hbm.at[pl.ds(grp * GROUP * CHUNK, GROUP * CHUNK)], rows_v)
            for j in range(GROUP):
                pltpu.sync_copy(rows_v.at[pl.ds(j * CHUNK, CHUNK)],
                                acc_sh.at[idx_v.at[j]], add=True)
                pltpu.sync_copy(ones_v, cnt_sh.at[idx_v.at[j]], add=True)
            return carry

        lax.fori_loop(0, n_mine, body, 0)
        plsc.subcore_barrier()

        r0 = s * (NUM_GRAPHS // NS)
        pltpu.sync_copy(acc_sh.at[pl.ds(r0, NUM_GRAPHS // NS)],
                        out_sum.at[c, pl.ds(r0, NUM_GRAPHS // NS)])
        pltpu.sync_copy(cnt_sh.at[pl.ds(r0, NUM_GRAPHS // NS)],
                        out_cnt.at[c, pl.ds(r0, NUM_GRAPHS // NS)])

    return k(node_rep, ids2d, zeros_sum, ones)


def _head(psum, pcnt, w, bvec):
    t = w.shape[1]

    def body(ps_ref, pc_ref, w_ref, b_ref, o_ref):
        seg = ps_ref[0] + ps_ref[1]
        cnt = (pc_ref[0] + pc_ref[1])[:, 0:1]
        rep = seg / jnp.maximum(cnt, 1.0)
        o_ref[...] = (
            lax.dot_general(rep, w_ref[...], (((1,), (0,)), ((), ())),
                            precision=lax.Precision.HIGHEST,
                            preferred_element_type=jnp.float32)
            + b_ref[...]
        )

    return pl.pallas_call(
        body,
        out_shape=jax.ShapeDtypeStruct((NUM_GRAPHS, t), jnp.float32),
    )(psum, pcnt, w, bvec.reshape(1, t))


def kernel(x, batch, W_gnn, b_gnn, W, b):
    n, d = x.shape
    n_pad = ((n + BLK - 1) // BLK) * BLK

    ids = jnp.concatenate(
        [batch.astype(jnp.int32),
         jnp.full((n_pad - n,), NUM_GRAPHS, jnp.int32)]
    ).reshape(-1, CHUNK)
    zeros_sum = jnp.zeros((NUM_GRAPHS + 1, d), jnp.float32)
    ones = jnp.ones((CHUNK, d), jnp.float32)

    node_rep = _gnn_matmul(x, W_gnn, b_gnn, n_pad)
    psum, pcnt = _sc_segment_sum(node_rep, ids, zeros_sum, ones)
    return _head(psum, pcnt, W, b)


# speedup vs baseline: 2.9125x; 1.0934x over previous
import functools

import jax
import jax.numpy as jnp
from jax import lax
from jax.experimental import pallas as pl
from jax.experimental.pallas import tpu as pltpu
from jax.experimental.pallas import tpu_sc as plsc

NUM_GRAPHS = 512
CHUNK = 128
GROUP = 2
BLK = 1024
NC, NS = 2, 16


def _gnn_matmul(x, w, bvec, n_pad):
    n, d = x.shape

    def body(x_ref, w_ref, b_ref, o_ref):
        acc = lax.dot_general(
            x_ref[...], w_ref[...], (((1,), (0,)), ((), ())),
            preferred_element_type=jnp.float32)
        o_ref[...] = jnp.maximum(acc + b_ref[...], 0.0)

    return pl.pallas_call(
        body,
        grid=(n_pad // BLK,),
        in_specs=[
            pl.BlockSpec((BLK, d), lambda i: (i, 0)),
            pl.BlockSpec((d, d), lambda i: (0, 0)),
            pl.BlockSpec((1, d), lambda i: (0, 0)),
        ],
        out_specs=pl.BlockSpec((BLK, d), lambda i: (i, 0)),
        out_shape=jax.ShapeDtypeStruct((n_pad, d), jnp.float32),
    )(x, w, bvec.reshape(1, d))


def _sc_segment_sum(node_rep, ids2d, zeros_sum, ones):
    d = node_rep.shape[1]
    n_groups = ids2d.shape[0] // GROUP
    nw = NC * NS
    mesh = plsc.VectorSubcoreMesh(core_axis_name="c", subcore_axis_name="s")

    @functools.partial(
        pl.kernel,
        out_type=[
            jax.ShapeDtypeStruct((NC, NUM_GRAPHS, d), jnp.float32),
            jax.ShapeDtypeStruct((NC, NUM_GRAPHS, d), jnp.float32),
        ],
        mesh=mesh,
        scratch_types=[
            pltpu.VMEM((GROUP, CHUNK), jnp.int32),
            pltpu.VMEM((GROUP * CHUNK, d), jnp.float32),
            pltpu.VMEM((CHUNK, d), jnp.float32),
            pltpu.VMEM_SHARED((NUM_GRAPHS + 1, d), jnp.float32),
            pltpu.VMEM_SHARED((NUM_GRAPHS + 1, d), jnp.float32),
        ],
    )
    def k(rep_hbm, ids_hbm, z_sum_hbm, ones_hbm, out_sum, out_cnt,
          idx_v, rows_v, ones_v, acc_sh, cnt_sh):
        c = lax.axis_index("c")
        s = lax.axis_index("s")
        wid = s * NC + c

        @pl.when(s == 0)
        def _zero():
            pltpu.sync_copy(z_sum_hbm, acc_sh)
            pltpu.sync_copy(z_sum_hbm, cnt_sh)

        pltpu.sync_copy(ones_hbm, ones_v)
        plsc.subcore_barrier()

        n_mine = (n_groups - wid + nw - 1) // nw

        def body(g, carry):
            grp = wid + g * nw
            pltpu.sync_copy(ids_hbm.at[pl.ds(grp * GROUP, GROUP)], idx_v)
            pltpu.sync_copy(
                rep_hbm.at[pl.ds(grp * GROUP * CHUNK, GROUP * CHUNK)], rows_v)
            for j in range(GROUP):
                pltpu.sync_copy(rows_v.at[pl.ds(j * CHUNK, CHUNK)],
                                acc_sh.at[idx_v.at[j]], add=True)
                pltpu.sync_copy(ones_v, cnt_sh.at[idx_v.at[j]], add=True)
            return carry

        lax.fori_loop(0, n_mine, body, 0)
        plsc.subcore_barrier()

        r0 = s * (NUM_GRAPHS // NS)
        pltpu.sync_copy(acc_sh.at[pl.ds(r0, NUM_GRAPHS // NS)],
                        out_sum.at[c, pl.ds(r0, NUM_GRAPHS // NS)])
        pltpu.sync_copy(cnt_sh.at[pl.ds(r0, NUM_GRAPHS // NS)],
                        out_cnt.at[c, pl.ds(r0, NUM_GRAPHS // NS)])

    return k(node_rep, ids2d, zeros_sum, ones)


def _head(psum, pcnt, w, bvec):
    t = w.shape[1]

    def body(ps_ref, pc_ref, w_ref, b_ref, o_ref):
        seg = ps_ref[0] + ps_ref[1]
        cnt = (pc_ref[0] + pc_ref[1])[:, 0:1]
        rep = seg / jnp.maximum(cnt, 1.0)
        o_ref[...] = (
            lax.dot_general(rep, w_ref[...], (((1,), (0,)), ((), ())),
                            precision=lax.Precision.HIGHEST,
                            preferred_element_type=jnp.float32)
            + b_ref[...]
        )

    return pl.pallas_call(
        body,
        out_shape=jax.ShapeDtypeStruct((NUM_GRAPHS, t), jnp.float32),
    )(psum, pcnt, w, bvec.reshape(1, t))


def kernel(x, batch, W_gnn, b_gnn, W, b):
    n, d = x.shape
    n_pad = ((n + BLK - 1) // BLK) * BLK

    ids = jnp.concatenate(
        [batch.astype(jnp.int32),
         jnp.full((n_pad - n,), NUM_GRAPHS, jnp.int32)]
    ).reshape(-1, CHUNK)
    zeros_sum = jnp.zeros((NUM_GRAPHS + 1, d), jnp.float32)
    ones = jnp.ones((CHUNK, d), jnp.float32)

    node_rep = _gnn_matmul(x, W_gnn, b_gnn, n_pad)
    psum, pcnt = _sc_segment_sum(node_rep, ids, zeros_sum, ones)
    return _head(psum, pcnt, W, b)
